# baseline (speedup 1.0000x reference)
import functools

import jax
import jax.numpy as jnp
from jax.experimental import pallas as pl
from jax.experimental.pallas import tpu as pltpu


def _gcn_block(adj_ref, x_ref, w_ref, out_ref, xw_ref):
    @pl.when(pl.program_id(0) == 0)
    def _():
        xw_ref[...] = jnp.dot(
            x_ref[...].astype(jnp.bfloat16),
            w_ref[...].astype(jnp.bfloat16),
            preferred_element_type=jnp.float32,
        ).astype(jnp.bfloat16)

    out_ref[...] = jnp.dot(
        adj_ref[...].astype(jnp.bfloat16),
        xw_ref[...],
        preferred_element_type=jnp.float32,
    )


@functools.partial(jax.jit, static_argnames=("block_m",))
def _gcn(inputs, adj, weight, block_m=400):
    n_rows, n_cols = adj.shape
    d_in = inputs.shape[1]
    d_out = weight.shape[1]
    grid = (n_rows // block_m,)
    return pl.pallas_call(
        _gcn_block,
        grid=grid,
        in_specs=[
            pl.BlockSpec((block_m, n_cols), lambda m: (m, 0)),
            pl.BlockSpec((n_cols, d_in), lambda m: (0, 0)),
            pl.BlockSpec((d_in, d_out), lambda m: (0, 0)),
        ],
        out_specs=pl.BlockSpec((block_m, d_out), lambda m: (m, 0)),
        out_shape=jax.ShapeDtypeStruct((n_rows, d_out), jnp.float32),
        scratch_shapes=[pltpu.VMEM((n_cols, d_out), jnp.bfloat16)],
    )(adj, inputs, weight)


def kernel(inputs, adj, weight):
    return _gcn(inputs, adj, weight)
